# baseline (speedup 1.0000x reference)
import functools

import jax
import jax.numpy as jnp
from jax import lax
from jax.experimental import pallas as pl
from jax.experimental.pallas import tpu as pltpu
from jax.experimental.pallas import tpu_sc as plsc

NUM_CORES = 2
NUM_SUBCORES = 16
NUM_WORKERS = NUM_CORES * NUM_SUBCORES
LANES = 16
SR = 8
DC = 512
LSLOTS = 2
OSLOTS = 2
UNROLL = 8


def kernel(x, embedding):
    B, S, D = x.shape
    s_per_w = S // NUM_WORKERS
    dh = D // DC
    chunks = (s_per_w // SR) * dh
    supers = chunks // LSLOTS

    mesh = plsc.VectorSubcoreMesh(core_axis_name="c", subcore_axis_name="s")

    scratch = []
    for _ in range(LSLOTS):
        scratch.append(pltpu.VMEM((SR, DC), jnp.float32))
        scratch.append(pltpu.VMEM((B, SR, DC), jnp.float32))
        scratch.append(pltpu.SemaphoreType.DMA)
    for _ in range(OSLOTS):
        scratch.append(pltpu.VMEM((B, SR, DC), jnp.float32))
        scratch.append(pltpu.SemaphoreType.DMA)

    @functools.partial(
        pl.kernel,
        mesh=mesh,
        out_type=jax.ShapeDtypeStruct((B, S, D), x.dtype),
        scratch_types=scratch,
        compiler_params=pltpu.CompilerParams(use_tc_tiling_on_sc=True),
    )
    def body(x_hbm, emb_hbm, out_hbm, *scr):
        lsets = [scr[3 * i:3 * i + 3] for i in range(LSLOTS)]
        obase = 3 * LSLOTS
        osets = [scr[obase + 2 * i:obase + 2 * i + 2] for i in range(OSLOTS)]
        wid = lax.axis_index("s") * NUM_CORES + lax.axis_index("c")
        s0 = wid * s_per_w

        def rowcol(gg):
            return s0 + (gg // dh) * SR, (gg % dh) * DC

        def issue_loads(ls, gg):
            eb, xb, lsem = lsets[ls]
            sr, dc = rowcol(gg)
            pltpu.async_copy(
                emb_hbm.at[pl.ds(sr, SR), pl.ds(dc, DC)], eb, lsem)
            pltpu.async_copy(
                x_hbm.at[pl.ds(0, B), pl.ds(sr, SR), pl.ds(dc, DC)],
                xb, lsem)

        def wait_loads(ls):
            eb, xb, lsem = lsets[ls]
            pltpu.make_async_copy(
                emb_hbm.at[pl.ds(0, SR), pl.ds(0, DC)], eb, lsem).wait()
            pltpu.make_async_copy(
                x_hbm.at[pl.ds(0, B), pl.ds(0, SR), pl.ds(0, DC)],
                xb, lsem).wait()

        def issue_stores(os_, gg):
            ob, ssem = osets[os_]
            sr, dc = rowcol(gg)
            pltpu.async_copy(
                ob, out_hbm.at[pl.ds(0, B), pl.ds(sr, SR), pl.ds(dc, DC)],
                ssem)

        def wait_stores(os_):
            ob, ssem = osets[os_]
            pltpu.make_async_copy(
                ob, out_hbm.at[pl.ds(0, B), pl.ds(0, SR), pl.ds(0, DC)],
                ssem).wait()

        def compute(ls, os_):
            eb, xb, _ = lsets[ls]
            ob = osets[os_][0]

            for r in range(SR):
                @plsc.parallel_loop(0, DC, step=LANES, unroll=UNROLL)
                def iter_body(i):
                    sl = pl.ds(i, LANES)
                    e = eb[r, sl]
                    for b in range(B):
                        ob[b, r, sl] = xb[b, r, sl] + e

        def chunk_body(gg, i, do_store_wait, do_load_issue):
            ls, os_ = i % LSLOTS, i % OSLOTS
            wait_loads(ls)
            if do_store_wait:
                wait_stores(os_)
            compute(ls, os_)
            if do_load_issue:
                issue_loads(ls, gg + LSLOTS)
            issue_stores(os_, gg)

        for g in range(LSLOTS):
            issue_loads(g, g)

        for i in range(LSLOTS):
            chunk_body(i, i, i >= OSLOTS, True)

        def super_body(s, c):
            base = s * LSLOTS
            for i in range(LSLOTS):
                chunk_body(base + i, i, True, True)
            return c

        lax.fori_loop(1, supers - 1, super_body, 0)

        base = (supers - 1) * LSLOTS
        for i in range(LSLOTS):
            chunk_body(base + i, i, True, False)

        for i in range(OSLOTS):
            wait_stores(i)

    return body(x, embedding)


# speedup vs baseline: 1.3062x; 1.3062x over previous
import functools

import jax
import jax.numpy as jnp
from jax import lax
from jax.experimental import pallas as pl
from jax.experimental.pallas import tpu as pltpu
from jax.experimental.pallas import tpu_sc as plsc

NUM_CORES = 2
NUM_SUBCORES = 16
NUM_WORKERS = NUM_CORES * NUM_SUBCORES
LANES = 16
SR = 8
DC = 512
LSLOTS = 4
OSLOTS = 2
UNROLL = 8


def kernel(x, embedding):
    B, S, D = x.shape
    s_per_w = S // NUM_WORKERS
    dh = D // DC
    chunks = (s_per_w // SR) * dh
    supers = chunks // LSLOTS

    mesh = plsc.VectorSubcoreMesh(core_axis_name="c", subcore_axis_name="s")

    scratch = []
    for _ in range(LSLOTS):
        scratch.append(pltpu.VMEM((SR, DC), jnp.float32))
        scratch.append(pltpu.VMEM((B, SR, DC), jnp.float32))
        scratch.append(pltpu.SemaphoreType.DMA)
    for _ in range(OSLOTS):
        scratch.append(pltpu.VMEM((B, SR, DC), jnp.float32))
        scratch.append(pltpu.SemaphoreType.DMA)

    @functools.partial(
        pl.kernel,
        mesh=mesh,
        out_type=jax.ShapeDtypeStruct((B, S, D), x.dtype),
        scratch_types=scratch,
        compiler_params=pltpu.CompilerParams(use_tc_tiling_on_sc=True),
    )
    def body(x_hbm, emb_hbm, out_hbm, *scr):
        lsets = [scr[3 * i:3 * i + 3] for i in range(LSLOTS)]
        obase = 3 * LSLOTS
        osets = [scr[obase + 2 * i:obase + 2 * i + 2] for i in range(OSLOTS)]
        wid = lax.axis_index("s") * NUM_CORES + lax.axis_index("c")
        s0 = wid * s_per_w

        def rowcol(gg):
            return s0 + (gg // dh) * SR, (gg % dh) * DC

        def issue_loads(ls, gg):
            eb, xb, lsem = lsets[ls]
            sr, dc = rowcol(gg)
            pltpu.async_copy(
                emb_hbm.at[pl.ds(sr, SR), pl.ds(dc, DC)], eb, lsem)
            pltpu.async_copy(
                x_hbm.at[pl.ds(0, B), pl.ds(sr, SR), pl.ds(dc, DC)],
                xb, lsem)

        def wait_loads(ls):
            eb, xb, lsem = lsets[ls]
            pltpu.make_async_copy(
                emb_hbm.at[pl.ds(0, SR), pl.ds(0, DC)], eb, lsem).wait()
            pltpu.make_async_copy(
                x_hbm.at[pl.ds(0, B), pl.ds(0, SR), pl.ds(0, DC)],
                xb, lsem).wait()

        def issue_stores(os_, gg):
            ob, ssem = osets[os_]
            sr, dc = rowcol(gg)
            pltpu.async_copy(
                ob, out_hbm.at[pl.ds(0, B), pl.ds(sr, SR), pl.ds(dc, DC)],
                ssem)

        def wait_stores(os_):
            ob, ssem = osets[os_]
            pltpu.make_async_copy(
                ob, out_hbm.at[pl.ds(0, B), pl.ds(0, SR), pl.ds(0, DC)],
                ssem).wait()

        def compute(ls, os_):
            eb, xb, _ = lsets[ls]
            ob = osets[os_][0]

            @plsc.parallel_loop(0, SR * DC, step=LANES, unroll=UNROLL)
            def iter_body(i):
                r = i // DC
                sl = pl.ds(i % DC, LANES)
                e = eb[r, sl]
                for b in range(B):
                    ob[b, r, sl] = xb[b, r, sl] + e

        def chunk_body(gg, i, do_store_wait, do_load_issue):
            ls, os_ = i % LSLOTS, i % OSLOTS
            wait_loads(ls)
            if do_load_issue:
                issue_loads((i + LSLOTS - 1) % LSLOTS, gg + LSLOTS - 1)
            if do_store_wait:
                wait_stores(os_)
            compute(ls, os_)
            issue_stores(os_, gg)

        for g in range(LSLOTS - 1):
            issue_loads(g, g)

        for i in range(LSLOTS):
            chunk_body(i, i, i >= OSLOTS, True)

        def super_body(s, c):
            base = s * LSLOTS
            for i in range(LSLOTS):
                chunk_body(base + i, i, True, True)
            return c

        lax.fori_loop(1, supers - 1, super_body, 0)

        base = (supers - 1) * LSLOTS
        for i in range(LSLOTS):
            gg = base + i
            chunk_body(gg, i, True, gg + LSLOTS - 1 < chunks)

        for i in range(OSLOTS):
            wait_stores(i)

    return body(x, embedding)
